# speedup vs baseline: 2.9787x; 2.9787x over previous
import functools

import jax
import jax.numpy as jnp
from jax import lax
from jax.experimental import pallas as pl
from jax.experimental.pallas import tpu as pltpu
from jax.experimental.pallas import tpu_sc as plsc

B = 4096
L = 50
DIM = 128
TOT = B * L
NC = 2
NS = 16
NW = NC * NS
PER_W = TOT // NW
CHUNK = 128
NCHUNK = PER_W // CHUNK


def _emb_body(ids_hbm, table_hbm, out_hbm, idx_v, rows_v, sem):
    wid = lax.axis_index("s") * NC + lax.axis_index("c")
    base = wid * PER_W
    pltpu.sync_copy(ids_hbm.at[wid], idx_v)

    def body(j, carry):
        pltpu.async_copy(table_hbm.at[idx_v.at[j]], rows_v, sem).wait()
        pltpu.sync_copy(rows_v, out_hbm.at[pl.ds(base + j * CHUNK, CHUNK)])
        return carry

    lax.fori_loop(0, NCHUNK, body, 0)


def kernel(word_ids, table):
    flat_ids = word_ids.reshape(NW, NCHUNK, CHUNK)
    mesh = plsc.VectorSubcoreMesh(core_axis_name="c", subcore_axis_name="s")
    emb = functools.partial(
        pl.kernel,
        mesh=mesh,
        out_type=jax.ShapeDtypeStruct((TOT, DIM), jnp.float32),
        scratch_types=[
            pltpu.VMEM((NCHUNK, CHUNK), jnp.int32),
            pltpu.VMEM((CHUNK, DIM), jnp.float32),
            pltpu.SemaphoreType.DMA,
        ],
    )(_emb_body)
    out = emb(flat_ids, table)
    return out.reshape(B, L, DIM)


# speedup vs baseline: 3.3136x; 1.1124x over previous
import functools

import jax
import jax.numpy as jnp
from jax import lax
from jax.experimental import pallas as pl
from jax.experimental.pallas import tpu as pltpu
from jax.experimental.pallas import tpu_sc as plsc

B = 4096
L = 50
DIM = 128
TOT = B * L
NC = 2
NS = 16
NW = NC * NS
PER_W = TOT // NW
CHUNK = 128
NCHUNK = PER_W // CHUNK
NBUF = 5
NGRP = NCHUNK // NBUF


def _emb_body(ids_hbm, table_hbm, out_hbm, idx_v, rows_v, *sems):
    gsems = sems[:NBUF]
    ssems = sems[NBUF:]
    wid = lax.axis_index("s") * NC + lax.axis_index("c")
    base = wid * PER_W
    pltpu.sync_copy(ids_hbm.at[wid], idx_v)

    def gstart(j, b):
        pltpu.make_async_copy(
            table_hbm.at[idx_v.at[j]], rows_v.at[b], gsems[b]).start()

    def gwait(b):
        pltpu.make_async_copy(
            table_hbm.at[idx_v.at[0]], rows_v.at[b], gsems[b]).wait()

    def sstart(j, b):
        pltpu.make_async_copy(
            rows_v.at[b], out_hbm.at[pl.ds(base + j * CHUNK, CHUNK)],
            ssems[b]).start()

    def swait(b):
        pltpu.make_async_copy(
            rows_v.at[b], out_hbm.at[pl.ds(base, CHUNK)], ssems[b]).wait()

    for b in range(NBUF):
        gstart(b, b)

    def body(g, carry):
        j0 = g * NBUF
        for b in range(NBUF):
            gwait(b)
            sstart(j0 + b, b)
        for b in range(NBUF):
            swait(b)
            gstart(j0 + NBUF + b, b)
        return carry

    lax.fori_loop(0, NGRP - 1, body, 0)

    j0 = (NGRP - 1) * NBUF
    for b in range(NBUF):
        gwait(b)
        sstart(j0 + b, b)
    for b in range(NBUF):
        swait(b)


def kernel(word_ids, table):
    flat_ids = word_ids.reshape(NW, NCHUNK, CHUNK)
    mesh = plsc.VectorSubcoreMesh(core_axis_name="c", subcore_axis_name="s")
    emb = functools.partial(
        pl.kernel,
        mesh=mesh,
        out_type=jax.ShapeDtypeStruct((TOT, DIM), jnp.float32),
        scratch_types=[
            pltpu.VMEM((NCHUNK, CHUNK), jnp.int32),
            pltpu.VMEM((NBUF, CHUNK, DIM), jnp.float32),
        ] + [pltpu.SemaphoreType.DMA] * (2 * NBUF),
    )(_emb_body)
    out = emb(flat_ids, table)
    return out.reshape(B, L, DIM)
